# Initial kernel scaffold; baseline (speedup 1.0000x reference)
import jax
import jax.numpy as jnp
from jax.experimental import pallas as pl
from jax.experimental.pallas import tpu as pltpu

N = 50000
E = 800000


def _scale_kernel(dinv_ref, h_ref, o_ref):
    o_ref[...] = h_ref[...] * dinv_ref[...]


def _scale(dinv, h):
    return pl.pallas_call(
        _scale_kernel,
        out_shape=jax.ShapeDtypeStruct(h.shape, h.dtype),
        grid=(h.shape[0] // 1000,),
        in_specs=[
            pl.BlockSpec((1000, 1), lambda i: (i, 0)),
            pl.BlockSpec((1000, h.shape[1]), lambda i: (i, 0)),
        ],
        out_specs=pl.BlockSpec((1000, h.shape[1]), lambda i: (i, 0)),
    )(dinv[:, None], h)


def kernel(z, edge_index, W1, b1, W2, b2):
    src = edge_index[0]
    dst = edge_index[1]
    ones = jnp.ones((E,), dtype=jnp.float32)
    deg = jax.ops.segment_sum(ones, dst, num_segments=N) + 1.0
    dinv = jax.lax.rsqrt(deg)

    def conv(x, W, b):
        h = x @ W
        hp = _scale(dinv, h)
        s = jax.ops.segment_sum(hp[src], dst, num_segments=N)
        return _scale(dinv, s + _scale(dinv, hp)) + b

    h = jax.nn.relu(conv(z, W1, b1))
    return conv(h, W2, b2)


# speedup vs baseline: 2.9447x; 2.9447x over previous
import jax
import jax.numpy as jnp
from jax.experimental import pallas as pl
from jax.experimental.pallas import tpu as pltpu

N = 50000
E = 800000


def _scale_kernel(dinv_ref, h_ref, o_ref):
    o_ref[...] = h_ref[...] * dinv_ref[...]


def _scale(dinv, h):
    return pl.pallas_call(
        _scale_kernel,
        out_shape=jax.ShapeDtypeStruct(h.shape, h.dtype),
        grid=(h.shape[0] // 1000,),
        in_specs=[
            pl.BlockSpec((1000, 1), lambda i: (i, 0)),
            pl.BlockSpec((1000, h.shape[1]), lambda i: (i, 0)),
        ],
        out_specs=pl.BlockSpec((1000, h.shape[1]), lambda i: (i, 0)),
    )(dinv[:, None], h)


def kernel(z, edge_index, W1, b1, W2, b2):
    src = edge_index[0]
    dst = edge_index[1]
    ones = jnp.ones((E,), dtype=jnp.float32)
    deg = jax.ops.segment_sum(ones, dst, num_segments=N) + 1.0
    dinv = jax.lax.rsqrt(deg)

    def conv(x, W, b):
        h = x @ W
        hp = _scale(dinv, h)
        s = jax.ops.segment_sum(hp[src], dst, num_segments=N)
        return _scale(dinv, s + _scale(dinv, h)) + b

    h = jax.nn.relu(conv(z, W1, b1))
    return conv(h, W2, b2)


# speedup vs baseline: 3.9308x; 1.3349x over previous
import dataclasses
import functools

import jax
import jax.numpy as jnp
from jax import lax
from jax.experimental import pallas as pl
from jax.experimental.pallas import tpu as pltpu
from jax.experimental.pallas import tpu_sc as plsc

N = 50000
E = 800000
D = 128
NC, NS = 2, 16
R = 784
NRC = 32
NPAD = NC * NRC * R
EPC = E + NRC * 64
WSC = 2000
NWIN = E // WSC
SB = 2048
WG = 64

_mesh = plsc.VectorSubcoreMesh(core_axis_name="c", subcore_axis_name="s")
_cp = pltpu.CompilerParams()
if "needs_layout_passes" in pltpu.CompilerParams.__dataclass_fields__:
    _cp = dataclasses.replace(_cp, needs_layout_passes=False)

_i0 = jnp.int32(0)


def _ceil64(x):
    return ((x + 63) // 64) * 64


def _k12_body(dst_hbm, src_hbm, counts_hbm, ebd_hbm, ebs_hbm, deg_hbm,
              win_d, win_s, st_d0, st_s0, st_d1, st_s1, deg0, deg1,
              cnt_v, call_v):
    c = lax.axis_index("c")
    s = lax.axis_index("s")
    lo0 = (c * NRC + s) * R
    lo1 = (c * NRC + 16 + s) * R
    zero_f = jnp.zeros((16,), jnp.float32)
    ones_f = jnp.ones((16,), jnp.float32)
    lane = lax.iota(jnp.int32, 16)

    @pl.loop(0, R + 16, step=16)
    def _(i):
        deg0[pl.ds(i, 16)] = zero_f
        deg1[pl.ds(i, 16)] = zero_f

    def win1(w, carry):
        pltpu.sync_copy(dst_hbm.at[pl.ds(pl.multiple_of(w * WSC, 8), WSC)], win_d)

        def vb(j, cc):
            c0, c1 = cc
            v = win_d[pl.ds(j * 16, 16)]
            v0 = v - lo0
            v1 = v - lo1
            m0 = (v0 >= 0) & (v0 < R)
            m1 = (v1 >= 0) & (v1 < R)
            plsc.addupdate_scatter(deg0, [jnp.where(m0, v0, R + lane)], ones_f)
            plsc.addupdate_scatter(deg1, [jnp.where(m1, v1, R + lane)], ones_f)
            return (c0 + jnp.sum(m0.astype(jnp.int32)),
                    c1 + jnp.sum(m1.astype(jnp.int32)))

        return lax.fori_loop(0, WSC // 16, vb, carry)

    c0, c1 = lax.fori_loop(0, NWIN, win1, (_i0, _i0))
    pc0 = _ceil64(c0)
    pc1 = _ceil64(c1)

    cnt_v[pl.ds(0, 16)] = jnp.full((16,), pc0, jnp.int32)
    pltpu.sync_copy(cnt_v, counts_hbm.at[pl.ds(pl.multiple_of((c * NRC + s) * 16, 16), 16)])
    cnt_v[pl.ds(0, 16)] = jnp.full((16,), pc1, jnp.int32)
    pltpu.sync_copy(cnt_v, counts_hbm.at[pl.ds(pl.multiple_of((c * NRC + s + 16) * 16, 16), 16)])
    plsc.subcore_barrier()

    pltpu.sync_copy(counts_hbm.at[pl.ds(pl.multiple_of(c * NRC * 16, 16), NRC * 16)], call_v)
    base0 = _i0
    base1 = _i0
    for i in range(NRC):
        pci = call_v[pl.ds(i * 16, 16)][0]
        base0 = base0 + jnp.where(i < s, pci, 0)
        base1 = base1 + jnp.where(i < s + 16, pci, 0)
    base0 = pl.multiple_of(base0, 64)
    base1 = pl.multiple_of(base1, 64)

    def win2(w, carry):
        pltpu.sync_copy(dst_hbm.at[pl.ds(pl.multiple_of(w * WSC, 8), WSC)], win_d)
        pltpu.sync_copy(src_hbm.at[pl.ds(pl.multiple_of(w * WSC, 8), WSC)], win_s)

        def vb(j, fc):
            f0, f1, d0, d1 = fc
            v = win_d[pl.ds(j * 16, 16)]
            u = win_s[pl.ds(j * 16, 16)]
            v0 = v - lo0
            v1 = v - lo1
            m0 = (v0 >= 0) & (v0 < R)
            m1 = (v1 >= 0) & (v1 < R)
            plsc.store_compressed(st_d0.at[pl.ds(f0, 16)], v0, mask=m0)
            plsc.store_compressed(st_s0.at[pl.ds(f0, 16)], u, mask=m0)
            plsc.store_compressed(st_d1.at[pl.ds(f1, 16)], v1, mask=m1)
            plsc.store_compressed(st_s1.at[pl.ds(f1, 16)], u, mask=m1)
            f0 = f0 + jnp.sum(m0.astype(jnp.int32))
            f1 = f1 + jnp.sum(m1.astype(jnp.int32))

            def flush0(args):
                f, d = args
                pltpu.sync_copy(st_d0.at[pl.ds(0, SB)],
                                ebd_hbm.at[pl.ds(pl.multiple_of(c * EPC + base0 + d, 64), SB)])
                pltpu.sync_copy(st_s0.at[pl.ds(0, SB)],
                                ebs_hbm.at[pl.ds(pl.multiple_of(c * EPC + base0 + d, 64), SB)])
                st_d0[pl.ds(0, 16)] = st_d0[pl.ds(SB, 16)]
                st_s0[pl.ds(0, 16)] = st_s0[pl.ds(SB, 16)]
                return f - SB, d + SB

            def flush1(args):
                f, d = args
                pltpu.sync_copy(st_d1.at[pl.ds(0, SB)],
                                ebd_hbm.at[pl.ds(pl.multiple_of(c * EPC + base1 + d, 64), SB)])
                pltpu.sync_copy(st_s1.at[pl.ds(0, SB)],
                                ebs_hbm.at[pl.ds(pl.multiple_of(c * EPC + base1 + d, 64), SB)])
                st_d1[pl.ds(0, 16)] = st_d1[pl.ds(SB, 16)]
                st_s1[pl.ds(0, 16)] = st_s1[pl.ds(SB, 16)]
                return f - SB, d + SB

            f0, d0 = lax.cond(f0 >= SB, flush0, lambda a: a, (f0, d0))
            f1, d1 = lax.cond(f1 >= SB, flush1, lambda a: a, (f1, d1))
            return f0, f1, d0, d1

        return lax.fori_loop(0, WSC // 16, vb, carry)

    f0, f1, d0, d1 = lax.fori_loop(0, NWIN, win2, (_i0, _i0, _i0, _i0))

    sent_d = jnp.zeros((16,), jnp.int32)
    sent_s = jnp.full((16,), N, jnp.int32)
    all_m = lane < 16
    for k in range(4):
        plsc.store_compressed(st_d0.at[pl.ds(f0 + k * 16, 16)], sent_d, mask=all_m)
        plsc.store_compressed(st_s0.at[pl.ds(f0 + k * 16, 16)], sent_s, mask=all_m)
        plsc.store_compressed(st_d1.at[pl.ds(f1 + k * 16, 16)], sent_d, mask=all_m)
        plsc.store_compressed(st_s1.at[pl.ds(f1 + k * 16, 16)], sent_s, mask=all_m)

    def fin0(k, _):
        pltpu.sync_copy(st_d0.at[pl.ds(k * 64, 64)],
                        ebd_hbm.at[pl.ds(pl.multiple_of(c * EPC + base0 + d0 + k * 64, 64), 64)])
        pltpu.sync_copy(st_s0.at[pl.ds(k * 64, 64)],
                        ebs_hbm.at[pl.ds(pl.multiple_of(c * EPC + base0 + d0 + k * 64, 64), 64)])
        return _i0

    def fin1(k, _):
        pltpu.sync_copy(st_d1.at[pl.ds(k * 64, 64)],
                        ebd_hbm.at[pl.ds(pl.multiple_of(c * EPC + base1 + d1 + k * 64, 64), 64)])
        pltpu.sync_copy(st_s1.at[pl.ds(k * 64, 64)],
                        ebs_hbm.at[pl.ds(pl.multiple_of(c * EPC + base1 + d1 + k * 64, 64), 64)])
        return _i0

    lax.fori_loop(0, (f0 + 63) // 64, fin0, _i0)
    lax.fori_loop(0, (f1 + 63) // 64, fin1, _i0)

    pltpu.sync_copy(deg0.at[pl.ds(0, R)], deg_hbm.at[pl.ds(pl.multiple_of(lo0, 16), R)])
    pltpu.sync_copy(deg1.at[pl.ds(0, R)], deg_hbm.at[pl.ds(pl.multiple_of(lo1, 16), R)])


def _k12(dst, src):
    return pl.kernel(
        _k12_body,
        out_type=(
            jax.ShapeDtypeStruct((NC * NRC * 16,), jnp.int32),
            jax.ShapeDtypeStruct((NC * EPC,), jnp.int32),
            jax.ShapeDtypeStruct((NC * EPC,), jnp.int32),
            jax.ShapeDtypeStruct((NPAD,), jnp.float32),
        ),
        mesh=_mesh,
        scratch_types=[
            pltpu.VMEM((WSC,), jnp.int32),
            pltpu.VMEM((WSC,), jnp.int32),
            pltpu.VMEM((SB + 80,), jnp.int32),
            pltpu.VMEM((SB + 80,), jnp.int32),
            pltpu.VMEM((SB + 80,), jnp.int32),
            pltpu.VMEM((SB + 80,), jnp.int32),
            pltpu.VMEM((R + 16,), jnp.float32),
            pltpu.VMEM((R + 16,), jnp.float32),
            pltpu.VMEM((16,), jnp.int32),
            pltpu.VMEM((NRC * 16,), jnp.int32),
        ],
        compiler_params=_cp,
    )(dst, src)


def _k3_body(hp_hbm, counts_hbm, ebd_hbm, ebs_hbm, s_hbm,
             call_v, acc_v, sidx, didx, rows_v, sem):
    c = lax.axis_index("c")
    s = lax.axis_index("s")
    zero_f = jnp.zeros((16,), jnp.float32)

    pltpu.sync_copy(counts_hbm.at[pl.ds(pl.multiple_of(c * NRC * 16, 16), NRC * 16)], call_v)

    base0 = _i0
    base1 = _i0
    for i in range(NRC):
        pci = call_v[pl.ds(i * 16, 16)][0]
        base0 = base0 + jnp.where(i < s, pci, 0)
        base1 = base1 + jnp.where(i < s + 16, pci, 0)
    base0 = pl.multiple_of(base0, 64)
    base1 = pl.multiple_of(base1, 64)
    pc0 = call_v[pl.ds(pl.multiple_of(s * 16, 16), 16)][0]
    pc1 = call_v[pl.ds(pl.multiple_of((s + 16) * 16, 16), 16)][0]

    for p in range(2):
        base = base0 if p == 0 else base1
        pc = pc0 if p == 0 else pc1
        rng = c * NRC + p * 16 + s

        @pl.loop(0, R)
        def _(i):
            for f in range(8):
                acc_v[i, pl.ds(f * 16, 16)] = zero_f

        def win(w, _):
            off = pl.multiple_of(base + w * WG, 64)
            pltpu.sync_copy(ebs_hbm.at[pl.ds(pl.multiple_of(c * EPC + off, 64), WG)], sidx)
            pltpu.sync_copy(ebd_hbm.at[pl.ds(pl.multiple_of(c * EPC + off, 64), WG)], didx)
            pltpu.async_copy(hp_hbm.at[sidx], rows_v, sem).wait()

            @pl.loop(0, WG, step=16)
            def _(j):
                dv = didx[pl.ds(j, 16)]
                for l in range(16):
                    d = dv[l]
                    for f in range(8):
                        plsc.addupdate(
                            acc_v.at[d, pl.ds(f * 16, 16)],
                            rows_v[j + l, pl.ds(f * 16, 16)],
                        )

            return _i0

        lax.fori_loop(0, pc // WG, win, _i0)
        pltpu.sync_copy(acc_v, s_hbm.at[pl.ds(pl.multiple_of(rng * R, 16), R), :])


def _k3(hp, counts, ebd, ebs):
    return pl.kernel(
        _k3_body,
        out_type=jax.ShapeDtypeStruct((NPAD, D), jnp.float32),
        mesh=_mesh,
        scratch_types=[
            pltpu.VMEM((NRC * 16,), jnp.int32),
            pltpu.VMEM((R, D), jnp.float32),
            pltpu.VMEM((WG,), jnp.int32),
            pltpu.VMEM((WG,), jnp.int32),
            pltpu.VMEM((WG, D), jnp.float32),
            pltpu.SemaphoreType.DMA,
        ],
    )(hp, counts, ebd, ebs)


_BM = 784
_GRID = NPAD // _BM


def _mm1_k(z_ref, w_ref, o_ref):
    o_ref[...] = lax.dot_general(
        z_ref[...], w_ref[...], (((1,), (0,)), ((), ())),
        precision=lax.Precision.HIGHEST, preferred_element_type=jnp.float32)


def _mm1(zp, W1):
    return pl.pallas_call(
        _mm1_k,
        out_shape=jax.ShapeDtypeStruct((NPAD, D), jnp.float32),
        grid=(_GRID,),
        in_specs=[
            pl.BlockSpec((_BM, 64), lambda i: (i, 0)),
            pl.BlockSpec((64, D), lambda i: (0, 0)),
        ],
        out_specs=pl.BlockSpec((_BM, D), lambda i: (i, 0)),
    )(zp, W1)


def _scale_k(deg_ref, h_ref, o_ref):
    dinv = lax.rsqrt(deg_ref[...] + 1.0)
    o_ref[...] = dinv * h_ref[...]


def _scale(deg2, h):
    return pl.pallas_call(
        _scale_k,
        out_shape=jax.ShapeDtypeStruct((NPAD, D), jnp.float32),
        grid=(_GRID,),
        in_specs=[
            pl.BlockSpec((_BM, 1), lambda i: (i, 0)),
            pl.BlockSpec((_BM, D), lambda i: (i, 0)),
        ],
        out_specs=pl.BlockSpec((_BM, D), lambda i: (i, 0)),
    )(deg2, h)


def _c1_k(deg_ref, s_ref, h1_ref, w2_ref, b1_ref, h2_ref, hp2_ref):
    i = pl.program_id(0)
    dinv = lax.rsqrt(deg_ref[...] + 1.0)
    t = dinv * s_ref[...] + dinv * dinv * h1_ref[...] + b1_ref[...]
    h = jnp.maximum(t, 0.0)
    h2 = lax.dot_general(
        h, w2_ref[...], (((1,), (0,)), ((), ())),
        precision=lax.Precision.HIGHEST, preferred_element_type=jnp.float32)
    h2_ref[...] = h2
    rid = i * _BM + lax.broadcasted_iota(jnp.int32, (_BM, 1), 0)
    hp2_ref[...] = jnp.where(rid < N, dinv * h2, 0.0)


def _c1(deg2, s1, h1, W2, b1):
    return pl.pallas_call(
        _c1_k,
        out_shape=(
            jax.ShapeDtypeStruct((NPAD, D), jnp.float32),
            jax.ShapeDtypeStruct((NPAD, D), jnp.float32),
        ),
        grid=(_GRID,),
        in_specs=[
            pl.BlockSpec((_BM, 1), lambda i: (i, 0)),
            pl.BlockSpec((_BM, D), lambda i: (i, 0)),
            pl.BlockSpec((_BM, D), lambda i: (i, 0)),
            pl.BlockSpec((D, D), lambda i: (0, 0)),
            pl.BlockSpec((1, D), lambda i: (0, 0)),
        ],
        out_specs=(
            pl.BlockSpec((_BM, D), lambda i: (i, 0)),
            pl.BlockSpec((_BM, D), lambda i: (i, 0)),
        ),
    )(deg2, s1, h1, W2, b1)


_BO = 2000


def _c2_k(deg_ref, s_ref, h2_ref, b2_ref, o_ref):
    dinv = lax.rsqrt(deg_ref[...] + 1.0)
    o_ref[...] = dinv * s_ref[...] + dinv * dinv * h2_ref[...] + b2_ref[...]


def _c2(deg2, s2, h2, b2):
    return pl.pallas_call(
        _c2_k,
        out_shape=jax.ShapeDtypeStruct((N, D), jnp.float32),
        grid=(N // _BO,),
        in_specs=[
            pl.BlockSpec((_BO, 1), lambda i: (i, 0)),
            pl.BlockSpec((_BO, D), lambda i: (i, 0)),
            pl.BlockSpec((_BO, D), lambda i: (i, 0)),
            pl.BlockSpec((1, D), lambda i: (0, 0)),
        ],
        out_specs=pl.BlockSpec((_BO, D), lambda i: (i, 0)),
    )(deg2, s2, h2, b2)


def kernel(z, edge_index, W1, b1, W2, b2):
    src = edge_index[0]
    dst = edge_index[1]
    zp = jnp.pad(z, ((0, NPAD - N), (0, 0)))

    counts, ebd, ebs, deg = _k12(dst, src)
    deg2 = deg[:, None]

    h1 = _mm1(zp, W1)
    hp1 = _scale(deg2, h1)
    s1 = _k3(hp1, counts, ebd, ebs)
    h2, hp2 = _c1(deg2, s1, h1, W2, b1[None, :])
    s2 = _k3(hp2, counts, ebd, ebs)
    return _c2(deg2, s2, h2, b2[None, :])


# speedup vs baseline: 4.9192x; 1.2514x over previous
import dataclasses
import functools

import jax
import jax.numpy as jnp
from jax import lax
from jax.experimental import pallas as pl
from jax.experimental.pallas import tpu as pltpu
from jax.experimental.pallas import tpu_sc as plsc

N = 50000
E = 800000
D = 128
NC, NS = 2, 16
R = 784
NRC = 32
NPAD = NC * NRC * R
EPC = E + NRC * 512
WSC = 2000
NWIN = E // WSC
SB = 2048
WG = 64

_mesh = plsc.VectorSubcoreMesh(core_axis_name="c", subcore_axis_name="s")
_cp = pltpu.CompilerParams()
if "needs_layout_passes" in pltpu.CompilerParams.__dataclass_fields__:
    _cp = dataclasses.replace(_cp, needs_layout_passes=False)

_i0 = jnp.int32(0)


def _ceil64(x):
    return ((x + 63) // 64) * 64


def _ceil512(x):
    return ((x + 511) // 512) * 512


def _k12_body(dst_hbm, src_hbm, counts_hbm, ebd_hbm, ebs_hbm, deg_hbm,
              win_d, win_s, st_d0, st_s0, st_d1, st_s1, deg0, deg1,
              cnt_v, call_v, sent_buf_d, sent_buf_s):
    c = lax.axis_index("c")
    s = lax.axis_index("s")
    lo0 = (c * NRC + s) * R
    lo1 = (c * NRC + 16 + s) * R
    zero_f = jnp.zeros((16,), jnp.float32)
    ones_f = jnp.ones((16,), jnp.float32)
    lane = lax.iota(jnp.int32, 16)

    @pl.loop(0, R + 16, step=16)
    def _(i):
        deg0[pl.ds(i, 16)] = zero_f
        deg1[pl.ds(i, 16)] = zero_f

    def win1(w, _):
        pltpu.sync_copy(dst_hbm.at[pl.ds(pl.multiple_of(w * WSC, 8), WSC)], win_d)

        def vb(j, carry):
            v = win_d[pl.ds(j * 16, 16)]
            v0 = v - lo0
            v1 = v - lo1
            m0 = (v0 >= 0) & (v0 < R)
            m1 = (v1 >= 0) & (v1 < R)
            plsc.addupdate_scatter(deg0, [jnp.where(m0, v0, R + lane)], ones_f)
            plsc.addupdate_scatter(deg1, [jnp.where(m1, v1, R + lane)], ones_f)
            return carry

        return lax.fori_loop(0, WSC // 16, vb, _i0)

    lax.fori_loop(0, NWIN, win1, _i0)
    acc0 = jnp.zeros((16,), jnp.float32)
    acc1 = jnp.zeros((16,), jnp.float32)
    for i in range(R // 16):
        acc0 = acc0 + deg0[pl.ds(i * 16, 16)]
        acc1 = acc1 + deg1[pl.ds(i * 16, 16)]
    c0 = jnp.sum(acc0).astype(jnp.int32)
    c1 = jnp.sum(acc1).astype(jnp.int32)
    pc0 = _ceil512(c0)
    pc1 = _ceil512(c1)

    cnt_v[pl.ds(0, 16)] = jnp.full((16,), pc0, jnp.int32)
    pltpu.sync_copy(cnt_v, counts_hbm.at[pl.ds(pl.multiple_of((c * NRC + s) * 16, 16), 16)])
    cnt_v[pl.ds(0, 16)] = jnp.full((16,), pc1, jnp.int32)
    pltpu.sync_copy(cnt_v, counts_hbm.at[pl.ds(pl.multiple_of((c * NRC + s + 16) * 16, 16), 16)])
    plsc.subcore_barrier()

    pltpu.sync_copy(counts_hbm.at[pl.ds(pl.multiple_of(c * NRC * 16, 16), NRC * 16)], call_v)
    base0 = _i0
    base1 = _i0
    for i in range(NRC):
        pci = call_v[pl.ds(i * 16, 16)][0]
        base0 = base0 + jnp.where(i < s, pci, 0)
        base1 = base1 + jnp.where(i < s + 16, pci, 0)
    base0 = pl.multiple_of(base0, 512)
    base1 = pl.multiple_of(base1, 512)

    def win2(w, carry):
        pltpu.sync_copy(dst_hbm.at[pl.ds(pl.multiple_of(w * WSC, 8), WSC)], win_d)
        pltpu.sync_copy(src_hbm.at[pl.ds(pl.multiple_of(w * WSC, 8), WSC)], win_s)

        def vb(j, fc):
            f0, f1, d0, d1 = fc
            v = win_d[pl.ds(j * 16, 16)]
            u = win_s[pl.ds(j * 16, 16)]
            v0 = v - lo0
            v1 = v - lo1
            m0 = (v0 >= 0) & (v0 < R)
            m1 = (v1 >= 0) & (v1 < R)
            plsc.store_compressed(st_d0.at[pl.ds(f0, 16)], v0, mask=m0)
            plsc.store_compressed(st_s0.at[pl.ds(f0, 16)], u, mask=m0)
            plsc.store_compressed(st_d1.at[pl.ds(f1, 16)], v1, mask=m1)
            plsc.store_compressed(st_s1.at[pl.ds(f1, 16)], u, mask=m1)
            f0 = f0 + plsc.all_reduce_population_count(m0)[0]
            f1 = f1 + plsc.all_reduce_population_count(m1)[0]

            def flush0(args):
                f, d = args
                pltpu.sync_copy(st_d0.at[pl.ds(0, SB)],
                                ebd_hbm.at[pl.ds(pl.multiple_of(c * EPC + base0 + d, 64), SB)])
                pltpu.sync_copy(st_s0.at[pl.ds(0, SB)],
                                ebs_hbm.at[pl.ds(pl.multiple_of(c * EPC + base0 + d, 64), SB)])
                st_d0[pl.ds(0, 16)] = st_d0[pl.ds(SB, 16)]
                st_s0[pl.ds(0, 16)] = st_s0[pl.ds(SB, 16)]
                return f - SB, d + SB

            def flush1(args):
                f, d = args
                pltpu.sync_copy(st_d1.at[pl.ds(0, SB)],
                                ebd_hbm.at[pl.ds(pl.multiple_of(c * EPC + base1 + d, 64), SB)])
                pltpu.sync_copy(st_s1.at[pl.ds(0, SB)],
                                ebs_hbm.at[pl.ds(pl.multiple_of(c * EPC + base1 + d, 64), SB)])
                st_d1[pl.ds(0, 16)] = st_d1[pl.ds(SB, 16)]
                st_s1[pl.ds(0, 16)] = st_s1[pl.ds(SB, 16)]
                return f - SB, d + SB

            f0, d0 = lax.cond(f0 >= SB, flush0, lambda a: a, (f0, d0))
            f1, d1 = lax.cond(f1 >= SB, flush1, lambda a: a, (f1, d1))
            return f0, f1, d0, d1

        return lax.fori_loop(0, WSC // 16, vb, carry)

    f0, f1, d0, d1 = lax.fori_loop(0, NWIN, win2, (_i0, _i0, _i0, _i0))

    sent_d = jnp.zeros((16,), jnp.int32)
    sent_s = jnp.full((16,), N, jnp.int32)
    all_m = lane < 16
    for k in range(4):
        plsc.store_compressed(st_d0.at[pl.ds(f0 + k * 16, 16)], sent_d, mask=all_m)
        plsc.store_compressed(st_s0.at[pl.ds(f0 + k * 16, 16)], sent_s, mask=all_m)
        plsc.store_compressed(st_d1.at[pl.ds(f1 + k * 16, 16)], sent_d, mask=all_m)
        plsc.store_compressed(st_s1.at[pl.ds(f1 + k * 16, 16)], sent_s, mask=all_m)

    def fin0(k, _):
        pltpu.sync_copy(st_d0.at[pl.ds(k * 64, 64)],
                        ebd_hbm.at[pl.ds(pl.multiple_of(c * EPC + base0 + d0 + k * 64, 64), 64)])
        pltpu.sync_copy(st_s0.at[pl.ds(k * 64, 64)],
                        ebs_hbm.at[pl.ds(pl.multiple_of(c * EPC + base0 + d0 + k * 64, 64), 64)])
        return _i0

    def fin1(k, _):
        pltpu.sync_copy(st_d1.at[pl.ds(k * 64, 64)],
                        ebd_hbm.at[pl.ds(pl.multiple_of(c * EPC + base1 + d1 + k * 64, 64), 64)])
        pltpu.sync_copy(st_s1.at[pl.ds(k * 64, 64)],
                        ebs_hbm.at[pl.ds(pl.multiple_of(c * EPC + base1 + d1 + k * 64, 64), 64)])
        return _i0

    lax.fori_loop(0, (f0 + 63) // 64, fin0, _i0)
    lax.fori_loop(0, (f1 + 63) // 64, fin1, _i0)

    for k in range(4):
        sent_buf_d[pl.ds(k * 16, 16)] = sent_d
        sent_buf_s[pl.ds(k * 16, 16)] = jnp.full((16,), N + k * 16, jnp.int32) + lane

    cov0 = d0 + ((f0 + 63) // 64) * 64
    cov1 = d1 + ((f1 + 63) // 64) * 64

    def pad0(k, _):
        pltpu.sync_copy(sent_buf_d,
                        ebd_hbm.at[pl.ds(pl.multiple_of(c * EPC + base0 + cov0 + k * 64, 64), 64)])
        pltpu.sync_copy(sent_buf_s,
                        ebs_hbm.at[pl.ds(pl.multiple_of(c * EPC + base0 + cov0 + k * 64, 64), 64)])
        return _i0

    def pad1(k, _):
        pltpu.sync_copy(sent_buf_d,
                        ebd_hbm.at[pl.ds(pl.multiple_of(c * EPC + base1 + cov1 + k * 64, 64), 64)])
        pltpu.sync_copy(sent_buf_s,
                        ebs_hbm.at[pl.ds(pl.multiple_of(c * EPC + base1 + cov1 + k * 64, 64), 64)])
        return _i0

    lax.fori_loop(0, (pc0 - cov0) // 64, pad0, _i0)
    lax.fori_loop(0, (pc1 - cov1) // 64, pad1, _i0)

    pltpu.sync_copy(deg0.at[pl.ds(0, R)], deg_hbm.at[pl.ds(pl.multiple_of(lo0, 16), R)])
    pltpu.sync_copy(deg1.at[pl.ds(0, R)], deg_hbm.at[pl.ds(pl.multiple_of(lo1, 16), R)])


def _k12(dst, src):
    return pl.kernel(
        _k12_body,
        out_type=(
            jax.ShapeDtypeStruct((NC * NRC * 16,), jnp.int32),
            jax.ShapeDtypeStruct((NC * EPC,), jnp.int32),
            jax.ShapeDtypeStruct((NC * EPC,), jnp.int32),
            jax.ShapeDtypeStruct((NPAD,), jnp.float32),
        ),
        mesh=_mesh,
        scratch_types=[
            pltpu.VMEM((WSC,), jnp.int32),
            pltpu.VMEM((WSC,), jnp.int32),
            pltpu.VMEM((SB + 80,), jnp.int32),
            pltpu.VMEM((SB + 80,), jnp.int32),
            pltpu.VMEM((SB + 80,), jnp.int32),
            pltpu.VMEM((SB + 80,), jnp.int32),
            pltpu.VMEM((R + 16,), jnp.float32),
            pltpu.VMEM((R + 16,), jnp.float32),
            pltpu.VMEM((16,), jnp.int32),
            pltpu.VMEM((NRC * 16,), jnp.int32),
            pltpu.VMEM((64,), jnp.int32),
            pltpu.VMEM((64,), jnp.int32),
        ],
        compiler_params=_cp,
    )(dst, src)


def _k3_body(hp_hbm, counts_hbm, ebd_hbm, ebs_hbm, s_hbm,
             call_v, acc_v, sidx, didx, rows0, rows1, sem0, sem1):
    c = lax.axis_index("c")
    s = lax.axis_index("s")
    zero_f = jnp.zeros((16,), jnp.float32)

    pltpu.sync_copy(counts_hbm.at[pl.ds(pl.multiple_of(c * NRC * 16, 16), NRC * 16)], call_v)

    base0 = _i0
    base1 = _i0
    for i in range(NRC):
        pci = call_v[pl.ds(i * 16, 16)][0]
        base0 = base0 + jnp.where(i < s, pci, 0)
        base1 = base1 + jnp.where(i < s + 16, pci, 0)
    base0 = pl.multiple_of(base0, 512)
    base1 = pl.multiple_of(base1, 512)
    pc0 = call_v[pl.ds(pl.multiple_of(s * 16, 16), 16)][0]
    pc1 = call_v[pl.ds(pl.multiple_of((s + 16) * 16, 16), 16)][0]

    bufs = (rows0, rows1)
    sems = (sem0, sem1)

    for p in range(2):
        base = base0 if p == 0 else base1
        pc = pc0 if p == 0 else pc1
        rng = c * NRC + p * 16 + s

        @pl.loop(0, R)
        def _(i):
            for f in range(8):
                acc_v[i, pl.ds(f * 16, 16)] = zero_f

        def chunk(ch, _):
            coff = pl.multiple_of(c * EPC + base + ch * 512, 512)
            pltpu.sync_copy(ebs_hbm.at[pl.ds(coff, 512)], sidx)
            pltpu.sync_copy(ebd_hbm.at[pl.ds(coff, 512)], didx)
            h = [None] * 8
            h[0] = pltpu.async_copy(hp_hbm.at[sidx.at[pl.ds(0, 64)]], rows0, sem0)
            for w in range(8):
                if w < 7:
                    h[w + 1] = pltpu.async_copy(
                        hp_hbm.at[sidx.at[pl.ds((w + 1) * 64, 64)]],
                        bufs[(w + 1) % 2], sems[(w + 1) % 2])
                h[w].wait()
                rows = bufs[w % 2]

                @pl.loop(0, 64, step=16)
                def _(j, w=w, rows=rows):
                    dv = didx[pl.ds(pl.multiple_of(w * 64 + j, 16), 16)]
                    for l in range(16):
                        d = dv[l]
                        for f in range(8):
                            plsc.addupdate(
                                acc_v.at[d, pl.ds(f * 16, 16)],
                                rows[j + l, pl.ds(f * 16, 16)],
                            )

            return _i0

        lax.fori_loop(0, pc // 512, chunk, _i0)
        pltpu.sync_copy(acc_v, s_hbm.at[pl.ds(pl.multiple_of(rng * R, 16), R), :])


def _k3(hp, counts, ebd, ebs):
    return pl.kernel(
        _k3_body,
        out_type=jax.ShapeDtypeStruct((NPAD, D), jnp.float32),
        mesh=_mesh,
        scratch_types=[
            pltpu.VMEM((NRC * 16,), jnp.int32),
            pltpu.VMEM((R, D), jnp.float32),
            pltpu.VMEM((512,), jnp.int32),
            pltpu.VMEM((512,), jnp.int32),
            pltpu.VMEM((WG, D), jnp.float32),
            pltpu.VMEM((WG, D), jnp.float32),
            pltpu.SemaphoreType.DMA,
            pltpu.SemaphoreType.DMA,
        ],
    )(hp, counts, ebd, ebs)


_BM = 784
_GRID = NPAD // _BM


def _mm1_k(z_ref, w_ref, o_ref):
    o_ref[...] = lax.dot_general(
        z_ref[...], w_ref[...], (((1,), (0,)), ((), ())),
        precision=lax.Precision.HIGHEST, preferred_element_type=jnp.float32)


def _mm1(zp, W1):
    return pl.pallas_call(
        _mm1_k,
        out_shape=jax.ShapeDtypeStruct((NPAD, D), jnp.float32),
        grid=(_GRID,),
        in_specs=[
            pl.BlockSpec((_BM, 64), lambda i: (i, 0)),
            pl.BlockSpec((64, D), lambda i: (0, 0)),
        ],
        out_specs=pl.BlockSpec((_BM, D), lambda i: (i, 0)),
    )(zp, W1)


def _scale_k(deg_ref, h_ref, o_ref):
    dinv = lax.rsqrt(deg_ref[...] + 1.0)
    o_ref[...] = dinv * h_ref[...]


def _scale(deg2, h):
    return pl.pallas_call(
        _scale_k,
        out_shape=jax.ShapeDtypeStruct((NPAD, D), jnp.float32),
        grid=(_GRID,),
        in_specs=[
            pl.BlockSpec((_BM, 1), lambda i: (i, 0)),
            pl.BlockSpec((_BM, D), lambda i: (i, 0)),
        ],
        out_specs=pl.BlockSpec((_BM, D), lambda i: (i, 0)),
    )(deg2, h)


def _c1_k(deg_ref, s_ref, h1_ref, w2_ref, b1_ref, h2_ref, hp2_ref):
    i = pl.program_id(0)
    dinv = lax.rsqrt(deg_ref[...] + 1.0)
    t = dinv * s_ref[...] + dinv * dinv * h1_ref[...] + b1_ref[...]
    h = jnp.maximum(t, 0.0)
    h2 = lax.dot_general(
        h, w2_ref[...], (((1,), (0,)), ((), ())),
        precision=lax.Precision.HIGHEST, preferred_element_type=jnp.float32)
    h2_ref[...] = h2
    rid = i * _BM + lax.broadcasted_iota(jnp.int32, (_BM, 1), 0)
    hp2_ref[...] = jnp.where(rid < N, dinv * h2, 0.0)


def _c1(deg2, s1, h1, W2, b1):
    return pl.pallas_call(
        _c1_k,
        out_shape=(
            jax.ShapeDtypeStruct((NPAD, D), jnp.float32),
            jax.ShapeDtypeStruct((NPAD, D), jnp.float32),
        ),
        grid=(_GRID,),
        in_specs=[
            pl.BlockSpec((_BM, 1), lambda i: (i, 0)),
            pl.BlockSpec((_BM, D), lambda i: (i, 0)),
            pl.BlockSpec((_BM, D), lambda i: (i, 0)),
            pl.BlockSpec((D, D), lambda i: (0, 0)),
            pl.BlockSpec((1, D), lambda i: (0, 0)),
        ],
        out_specs=(
            pl.BlockSpec((_BM, D), lambda i: (i, 0)),
            pl.BlockSpec((_BM, D), lambda i: (i, 0)),
        ),
    )(deg2, s1, h1, W2, b1)


_BO = 2000


def _c2_k(deg_ref, s_ref, h2_ref, b2_ref, o_ref):
    dinv = lax.rsqrt(deg_ref[...] + 1.0)
    o_ref[...] = dinv * s_ref[...] + dinv * dinv * h2_ref[...] + b2_ref[...]


def _c2(deg2, s2, h2, b2):
    return pl.pallas_call(
        _c2_k,
        out_shape=jax.ShapeDtypeStruct((N, D), jnp.float32),
        grid=(N // _BO,),
        in_specs=[
            pl.BlockSpec((_BO, 1), lambda i: (i, 0)),
            pl.BlockSpec((_BO, D), lambda i: (i, 0)),
            pl.BlockSpec((_BO, D), lambda i: (i, 0)),
            pl.BlockSpec((1, D), lambda i: (0, 0)),
        ],
        out_specs=pl.BlockSpec((_BO, D), lambda i: (i, 0)),
    )(deg2, s2, h2, b2)


def kernel(z, edge_index, W1, b1, W2, b2):
    src = edge_index[0]
    dst = edge_index[1]
    zp = jnp.pad(z, ((0, NPAD - N), (0, 0)))

    counts, ebd, ebs, deg = _k12(dst, src)
    deg2 = deg[:, None]

    h1 = _mm1(zp, W1)
    hp1 = _scale(deg2, h1)
    s1 = _k3(hp1, counts, ebd, ebs)
    h2, hp2 = _c1(deg2, s1, h1, W2, b1[None, :])
    s2 = _k3(hp2, counts, ebd, ebs)
    return _c2(deg2, s2, h2, b2[None, :])


# speedup vs baseline: 6.0492x; 1.2297x over previous
import dataclasses
import functools

import jax
import jax.numpy as jnp
from jax import lax
from jax.experimental import pallas as pl
from jax.experimental.pallas import tpu as pltpu
from jax.experimental.pallas import tpu_sc as plsc

N = 50000
E = 800000
D = 128
NC, NS = 2, 16
R = 784
NRC = 32
NPAD = NC * NRC * R
EPC = E + NRC * 512
WSC = 3200
NWIN = E // WSC
SB = 2048
STG = SB + 192
WG = 64

_mesh = plsc.VectorSubcoreMesh(core_axis_name="c", subcore_axis_name="s")
_cp = pltpu.CompilerParams()
if "needs_layout_passes" in pltpu.CompilerParams.__dataclass_fields__:
    _cp = dataclasses.replace(_cp, needs_layout_passes=False)

_i0 = jnp.int32(0)


def _ceil64(x):
    return ((x + 63) // 64) * 64


def _ceil512(x):
    return ((x + 511) // 512) * 512


def _k12_body(dst_hbm, src_hbm, counts_hbm, ebd_hbm, ebs_hbm, deg_hbm,
              win_d, win_s, st_d0, st_s0, st_d1, st_s1,
              d0a, d0b, d0c, d0d, d1a, d1b, d1c, d1d,
              cnt_v, call_v, sent_buf_d, sent_buf_s):
    c = lax.axis_index("c")
    s = lax.axis_index("s")
    lo0 = (c * NRC + s) * R
    lo1 = (c * NRC + 16 + s) * R
    zero_f = jnp.zeros((16,), jnp.float32)
    ones_f = jnp.ones((16,), jnp.float32)
    lane = lax.iota(jnp.int32, 16)

    deg0s = (d0a, d0b, d0c, d0d)
    deg1s = (d1a, d1b, d1c, d1d)

    @pl.loop(0, R + 16, step=16)
    def _(i):
        for k in range(4):
            deg0s[k][pl.ds(i, 16)] = zero_f
            deg1s[k][pl.ds(i, 16)] = zero_f

    def win1(w, _):
        pltpu.sync_copy(dst_hbm.at[pl.ds(pl.multiple_of(w * WSC, 8), WSC)], win_d)

        def vb4(j, carry):
            for k in range(4):
                v = win_d[pl.ds(pl.multiple_of(j * 64, 16) + k * 16, 16)]
                v0 = v - lo0
                v1 = v - lo1
                m0 = (v0 >= 0) & (v0 < R)
                m1 = (v1 >= 0) & (v1 < R)
                plsc.addupdate_scatter(deg0s[k], [jnp.where(m0, v0, R + lane)], ones_f)
                plsc.addupdate_scatter(deg1s[k], [jnp.where(m1, v1, R + lane)], ones_f)
            return carry

        return lax.fori_loop(0, WSC // 64, vb4, _i0)

    lax.fori_loop(0, NWIN, win1, _i0)
    acc0 = jnp.zeros((16,), jnp.float32)
    acc1 = jnp.zeros((16,), jnp.float32)
    for i in range(R // 16):
        m0v = (d0a[pl.ds(i * 16, 16)] + d0b[pl.ds(i * 16, 16)]
               + d0c[pl.ds(i * 16, 16)] + d0d[pl.ds(i * 16, 16)])
        m1v = (d1a[pl.ds(i * 16, 16)] + d1b[pl.ds(i * 16, 16)]
               + d1c[pl.ds(i * 16, 16)] + d1d[pl.ds(i * 16, 16)])
        d0a[pl.ds(i * 16, 16)] = m0v
        d1a[pl.ds(i * 16, 16)] = m1v
        acc0 = acc0 + m0v
        acc1 = acc1 + m1v
    c0 = jnp.sum(acc0).astype(jnp.int32)
    c1 = jnp.sum(acc1).astype(jnp.int32)
    pc0 = _ceil512(c0)
    pc1 = _ceil512(c1)

    cnt_v[pl.ds(0, 16)] = jnp.full((16,), pc0, jnp.int32)
    pltpu.sync_copy(cnt_v, counts_hbm.at[pl.ds(pl.multiple_of((c * NRC + s) * 16, 16), 16)])
    cnt_v[pl.ds(0, 16)] = jnp.full((16,), pc1, jnp.int32)
    pltpu.sync_copy(cnt_v, counts_hbm.at[pl.ds(pl.multiple_of((c * NRC + s + 16) * 16, 16), 16)])
    plsc.subcore_barrier()

    pltpu.sync_copy(counts_hbm.at[pl.ds(pl.multiple_of(c * NRC * 16, 16), NRC * 16)], call_v)
    base0 = _i0
    base1 = _i0
    for i in range(NRC):
        pci = call_v[pl.ds(i * 16, 16)][0]
        base0 = base0 + jnp.where(i < s, pci, 0)
        base1 = base1 + jnp.where(i < s + 16, pci, 0)
    base0 = pl.multiple_of(base0, 512)
    base1 = pl.multiple_of(base1, 512)

    def win2(w, carry):
        pltpu.sync_copy(dst_hbm.at[pl.ds(pl.multiple_of(w * WSC, 8), WSC)], win_d)
        pltpu.sync_copy(src_hbm.at[pl.ds(pl.multiple_of(w * WSC, 8), WSC)], win_s)

        def vb4(j, fc):
            f0, f1, d0, d1 = fc
            for k in range(4):
                off = pl.multiple_of(j * 64, 16) + k * 16
                v = win_d[pl.ds(off, 16)]
                u = win_s[pl.ds(off, 16)]
                v0 = v - lo0
                v1 = v - lo1
                m0 = (v0 >= 0) & (v0 < R)
                m1 = (v1 >= 0) & (v1 < R)
                plsc.store_compressed(st_d0.at[pl.ds(f0, 16)], v0, mask=m0)
                plsc.store_compressed(st_s0.at[pl.ds(f0, 16)], u, mask=m0)
                plsc.store_compressed(st_d1.at[pl.ds(f1, 16)], v1, mask=m1)
                plsc.store_compressed(st_s1.at[pl.ds(f1, 16)], u, mask=m1)
                f0 = f0 + plsc.all_reduce_population_count(m0)[0]
                f1 = f1 + plsc.all_reduce_population_count(m1)[0]

            def flush0(args):
                f, d = args
                pltpu.sync_copy(st_d0.at[pl.ds(0, SB)],
                                ebd_hbm.at[pl.ds(pl.multiple_of(c * EPC + base0 + d, 64), SB)])
                pltpu.sync_copy(st_s0.at[pl.ds(0, SB)],
                                ebs_hbm.at[pl.ds(pl.multiple_of(c * EPC + base0 + d, 64), SB)])
                for t in range(8):
                    st_d0[pl.ds(t * 16, 16)] = st_d0[pl.ds(SB + t * 16, 16)]
                    st_s0[pl.ds(t * 16, 16)] = st_s0[pl.ds(SB + t * 16, 16)]
                return f - SB, d + SB

            def flush1(args):
                f, d = args
                pltpu.sync_copy(st_d1.at[pl.ds(0, SB)],
                                ebd_hbm.at[pl.ds(pl.multiple_of(c * EPC + base1 + d, 64), SB)])
                pltpu.sync_copy(st_s1.at[pl.ds(0, SB)],
                                ebs_hbm.at[pl.ds(pl.multiple_of(c * EPC + base1 + d, 64), SB)])
                for t in range(8):
                    st_d1[pl.ds(t * 16, 16)] = st_d1[pl.ds(SB + t * 16, 16)]
                    st_s1[pl.ds(t * 16, 16)] = st_s1[pl.ds(SB + t * 16, 16)]
                return f - SB, d + SB

            f0, d0 = lax.cond(f0 >= SB, flush0, lambda a: a, (f0, d0))
            f1, d1 = lax.cond(f1 >= SB, flush1, lambda a: a, (f1, d1))
            return f0, f1, d0, d1

        return lax.fori_loop(0, WSC // 64, vb4, carry)

    f0, f1, d0, d1 = lax.fori_loop(0, NWIN, win2, (_i0, _i0, _i0, _i0))

    sent_d = jnp.zeros((16,), jnp.int32)
    sent_s = jnp.full((16,), N, jnp.int32)
    all_m = lane < 16
    for k in range(4):
        plsc.store_compressed(st_d0.at[pl.ds(f0 + k * 16, 16)], sent_d, mask=all_m)
        plsc.store_compressed(st_s0.at[pl.ds(f0 + k * 16, 16)], sent_s, mask=all_m)
        plsc.store_compressed(st_d1.at[pl.ds(f1 + k * 16, 16)], sent_d, mask=all_m)
        plsc.store_compressed(st_s1.at[pl.ds(f1 + k * 16, 16)], sent_s, mask=all_m)

    def fin0(k, _):
        pltpu.sync_copy(st_d0.at[pl.ds(k * 64, 64)],
                        ebd_hbm.at[pl.ds(pl.multiple_of(c * EPC + base0 + d0 + k * 64, 64), 64)])
        pltpu.sync_copy(st_s0.at[pl.ds(k * 64, 64)],
                        ebs_hbm.at[pl.ds(pl.multiple_of(c * EPC + base0 + d0 + k * 64, 64), 64)])
        return _i0

    def fin1(k, _):
        pltpu.sync_copy(st_d1.at[pl.ds(k * 64, 64)],
                        ebd_hbm.at[pl.ds(pl.multiple_of(c * EPC + base1 + d1 + k * 64, 64), 64)])
        pltpu.sync_copy(st_s1.at[pl.ds(k * 64, 64)],
                        ebs_hbm.at[pl.ds(pl.multiple_of(c * EPC + base1 + d1 + k * 64, 64), 64)])
        return _i0

    lax.fori_loop(0, (f0 + 63) // 64, fin0, _i0)
    lax.fori_loop(0, (f1 + 63) // 64, fin1, _i0)

    for k in range(4):
        sent_buf_d[pl.ds(k * 16, 16)] = sent_d
        sent_buf_s[pl.ds(k * 16, 16)] = jnp.full((16,), N + k * 16, jnp.int32) + lane

    cov0 = d0 + ((f0 + 63) // 64) * 64
    cov1 = d1 + ((f1 + 63) // 64) * 64

    def pad0(k, _):
        pltpu.sync_copy(sent_buf_d,
                        ebd_hbm.at[pl.ds(pl.multiple_of(c * EPC + base0 + cov0 + k * 64, 64), 64)])
        pltpu.sync_copy(sent_buf_s,
                        ebs_hbm.at[pl.ds(pl.multiple_of(c * EPC + base0 + cov0 + k * 64, 64), 64)])
        return _i0

    def pad1(k, _):
        pltpu.sync_copy(sent_buf_d,
                        ebd_hbm.at[pl.ds(pl.multiple_of(c * EPC + base1 + cov1 + k * 64, 64), 64)])
        pltpu.sync_copy(sent_buf_s,
                        ebs_hbm.at[pl.ds(pl.multiple_of(c * EPC + base1 + cov1 + k * 64, 64), 64)])
        return _i0

    lax.fori_loop(0, (pc0 - cov0) // 64, pad0, _i0)
    lax.fori_loop(0, (pc1 - cov1) // 64, pad1, _i0)

    pltpu.sync_copy(d0a.at[pl.ds(0, R)], deg_hbm.at[pl.ds(pl.multiple_of(lo0, 16), R)])
    pltpu.sync_copy(d1a.at[pl.ds(0, R)], deg_hbm.at[pl.ds(pl.multiple_of(lo1, 16), R)])


def _k12(dst, src):
    return pl.kernel(
        _k12_body,
        out_type=(
            jax.ShapeDtypeStruct((NC * NRC * 16,), jnp.int32),
            jax.ShapeDtypeStruct((NC * EPC,), jnp.int32),
            jax.ShapeDtypeStruct((NC * EPC,), jnp.int32),
            jax.ShapeDtypeStruct((NPAD,), jnp.float32),
        ),
        mesh=_mesh,
        scratch_types=[
            pltpu.VMEM((WSC,), jnp.int32),
            pltpu.VMEM((WSC,), jnp.int32),
            pltpu.VMEM((STG,), jnp.int32),
            pltpu.VMEM((STG,), jnp.int32),
            pltpu.VMEM((STG,), jnp.int32),
            pltpu.VMEM((STG,), jnp.int32),
            pltpu.VMEM((R + 16,), jnp.float32),
            pltpu.VMEM((R + 16,), jnp.float32),
            pltpu.VMEM((R + 16,), jnp.float32),
            pltpu.VMEM((R + 16,), jnp.float32),
            pltpu.VMEM((R + 16,), jnp.float32),
            pltpu.VMEM((R + 16,), jnp.float32),
            pltpu.VMEM((R + 16,), jnp.float32),
            pltpu.VMEM((R + 16,), jnp.float32),
            pltpu.VMEM((16,), jnp.int32),
            pltpu.VMEM((NRC * 16,), jnp.int32),
            pltpu.VMEM((64,), jnp.int32),
            pltpu.VMEM((64,), jnp.int32),
        ],
        compiler_params=_cp,
    )(dst, src)


def _k3_body(hp_hbm, counts_hbm, ebd_hbm, ebs_hbm, s_hbm,
             call_v, acc_v, sidx, didx, rows0, rows1, sem0, sem1):
    c = lax.axis_index("c")
    s = lax.axis_index("s")
    zero_f = jnp.zeros((16,), jnp.float32)

    pltpu.sync_copy(counts_hbm.at[pl.ds(pl.multiple_of(c * NRC * 16, 16), NRC * 16)], call_v)

    base0 = _i0
    base1 = _i0
    for i in range(NRC):
        pci = call_v[pl.ds(i * 16, 16)][0]
        base0 = base0 + jnp.where(i < s, pci, 0)
        base1 = base1 + jnp.where(i < s + 16, pci, 0)
    base0 = pl.multiple_of(base0, 512)
    base1 = pl.multiple_of(base1, 512)
    pc0 = call_v[pl.ds(pl.multiple_of(s * 16, 16), 16)][0]
    pc1 = call_v[pl.ds(pl.multiple_of((s + 16) * 16, 16), 16)][0]

    bufs = (rows0, rows1)
    sems = (sem0, sem1)

    for p in range(2):
        base = base0 if p == 0 else base1
        pc = pc0 if p == 0 else pc1
        rng = c * NRC + p * 16 + s

        @pl.loop(0, R)
        def _(i):
            for f in range(8):
                acc_v[i, pl.ds(f * 16, 16)] = zero_f

        def chunk(ch, _):
            coff = pl.multiple_of(c * EPC + base + ch * 512, 512)
            pltpu.sync_copy(ebs_hbm.at[pl.ds(coff, 512)], sidx)
            pltpu.sync_copy(ebd_hbm.at[pl.ds(coff, 512)], didx)
            h = [None] * 8
            h[0] = pltpu.async_copy(hp_hbm.at[sidx.at[pl.ds(0, 64)]], rows0, sem0)
            for w in range(8):
                if w < 7:
                    h[w + 1] = pltpu.async_copy(
                        hp_hbm.at[sidx.at[pl.ds((w + 1) * 64, 64)]],
                        bufs[(w + 1) % 2], sems[(w + 1) % 2])
                h[w].wait()
                rows = bufs[w % 2]

                @pl.loop(0, 64, step=16)
                def _(j, w=w, rows=rows):
                    dv = didx[pl.ds(pl.multiple_of(w * 64 + j, 16), 16)]
                    for l in range(16):
                        d = dv[l]
                        for f in range(8):
                            plsc.addupdate(
                                acc_v.at[d, pl.ds(f * 16, 16)],
                                rows[j + l, pl.ds(f * 16, 16)],
                            )

            return _i0

        lax.fori_loop(0, pc // 512, chunk, _i0)
        pltpu.sync_copy(acc_v, s_hbm.at[pl.ds(pl.multiple_of(rng * R, 16), R), :])


def _k3(hp, counts, ebd, ebs):
    return pl.kernel(
        _k3_body,
        out_type=jax.ShapeDtypeStruct((NPAD, D), jnp.float32),
        mesh=_mesh,
        scratch_types=[
            pltpu.VMEM((NRC * 16,), jnp.int32),
            pltpu.VMEM((R, D), jnp.float32),
            pltpu.VMEM((512,), jnp.int32),
            pltpu.VMEM((512,), jnp.int32),
            pltpu.VMEM((WG, D), jnp.float32),
            pltpu.VMEM((WG, D), jnp.float32),
            pltpu.SemaphoreType.DMA,
            pltpu.SemaphoreType.DMA,
        ],
    )(hp, counts, ebd, ebs)


_BM = 784
_GRID = NPAD // _BM


def _mm1_k(z_ref, w_ref, o_ref):
    o_ref[...] = lax.dot_general(
        z_ref[...], w_ref[...], (((1,), (0,)), ((), ())),
        precision=lax.Precision.HIGHEST, preferred_element_type=jnp.float32)


def _mm1(zp, W1):
    return pl.pallas_call(
        _mm1_k,
        out_shape=jax.ShapeDtypeStruct((NPAD, D), jnp.float32),
        grid=(_GRID,),
        in_specs=[
            pl.BlockSpec((_BM, 64), lambda i: (i, 0)),
            pl.BlockSpec((64, D), lambda i: (0, 0)),
        ],
        out_specs=pl.BlockSpec((_BM, D), lambda i: (i, 0)),
    )(zp, W1)


def _scale_k(deg_ref, h_ref, o_ref):
    dinv = lax.rsqrt(deg_ref[...] + 1.0)
    o_ref[...] = dinv * h_ref[...]


def _scale(deg2, h):
    return pl.pallas_call(
        _scale_k,
        out_shape=jax.ShapeDtypeStruct((NPAD, D), jnp.float32),
        grid=(_GRID,),
        in_specs=[
            pl.BlockSpec((_BM, 1), lambda i: (i, 0)),
            pl.BlockSpec((_BM, D), lambda i: (i, 0)),
        ],
        out_specs=pl.BlockSpec((_BM, D), lambda i: (i, 0)),
    )(deg2, h)


def _c1_k(deg_ref, s_ref, h1_ref, w2_ref, b1_ref, h2_ref, hp2_ref):
    i = pl.program_id(0)
    dinv = lax.rsqrt(deg_ref[...] + 1.0)
    t = dinv * s_ref[...] + dinv * dinv * h1_ref[...] + b1_ref[...]
    h = jnp.maximum(t, 0.0)
    h2 = lax.dot_general(
        h, w2_ref[...], (((1,), (0,)), ((), ())),
        precision=lax.Precision.HIGHEST, preferred_element_type=jnp.float32)
    h2_ref[...] = h2
    rid = i * _BM + lax.broadcasted_iota(jnp.int32, (_BM, 1), 0)
    hp2_ref[...] = jnp.where(rid < N, dinv * h2, 0.0)


def _c1(deg2, s1, h1, W2, b1):
    return pl.pallas_call(
        _c1_k,
        out_shape=(
            jax.ShapeDtypeStruct((NPAD, D), jnp.float32),
            jax.ShapeDtypeStruct((NPAD, D), jnp.float32),
        ),
        grid=(_GRID,),
        in_specs=[
            pl.BlockSpec((_BM, 1), lambda i: (i, 0)),
            pl.BlockSpec((_BM, D), lambda i: (i, 0)),
            pl.BlockSpec((_BM, D), lambda i: (i, 0)),
            pl.BlockSpec((D, D), lambda i: (0, 0)),
            pl.BlockSpec((1, D), lambda i: (0, 0)),
        ],
        out_specs=(
            pl.BlockSpec((_BM, D), lambda i: (i, 0)),
            pl.BlockSpec((_BM, D), lambda i: (i, 0)),
        ),
    )(deg2, s1, h1, W2, b1)


_BO = 2000


def _c2_k(deg_ref, s_ref, h2_ref, b2_ref, o_ref):
    dinv = lax.rsqrt(deg_ref[...] + 1.0)
    o_ref[...] = dinv * s_ref[...] + dinv * dinv * h2_ref[...] + b2_ref[...]


def _c2(deg2, s2, h2, b2):
    return pl.pallas_call(
        _c2_k,
        out_shape=jax.ShapeDtypeStruct((N, D), jnp.float32),
        grid=(N // _BO,),
        in_specs=[
            pl.BlockSpec((_BO, 1), lambda i: (i, 0)),
            pl.BlockSpec((_BO, D), lambda i: (i, 0)),
            pl.BlockSpec((_BO, D), lambda i: (i, 0)),
            pl.BlockSpec((1, D), lambda i: (0, 0)),
        ],
        out_specs=pl.BlockSpec((_BO, D), lambda i: (i, 0)),
    )(deg2, s2, h2, b2)


def kernel(z, edge_index, W1, b1, W2, b2):
    src = edge_index[0]
    dst = edge_index[1]
    zp = jnp.pad(z, ((0, NPAD - N), (0, 0)))

    counts, ebd, ebs, deg = _k12(dst, src)
    deg2 = deg[:, None]

    h1 = _mm1(zp, W1)
    hp1 = _scale(deg2, h1)
    s1 = _k3(hp1, counts, ebd, ebs)
    h2, hp2 = _c1(deg2, s1, h1, W2, b1[None, :])
    s2 = _k3(hp2, counts, ebd, ebs)
    return _c2(deg2, s2, h2, b2[None, :])


# speedup vs baseline: 6.9442x; 1.1479x over previous
import dataclasses
import functools

import jax
import jax.numpy as jnp
from jax import lax
from jax.experimental import pallas as pl
from jax.experimental.pallas import tpu as pltpu
from jax.experimental.pallas import tpu_sc as plsc

N = 50000
E = 800000
D = 128
NC, NS = 2, 16
R = 784
NRC = 32
NPAD = NC * NRC * R
EPC = E + NRC * 512
WSC = 3200
NWIN = E // WSC
SB = 2048
STG = SB + 192
WG = 64

_mesh = plsc.VectorSubcoreMesh(core_axis_name="c", subcore_axis_name="s")
_cp = pltpu.CompilerParams()
if "needs_layout_passes" in pltpu.CompilerParams.__dataclass_fields__:
    _cp = dataclasses.replace(_cp, needs_layout_passes=False)

_i0 = jnp.int32(0)


def _ceil64(x):
    return ((x + 63) // 64) * 64


def _ceil512(x):
    return ((x + 511) // 512) * 512


def _k12_body(dst_hbm, src_hbm, counts_hbm, ebd_hbm, ebs_hbm, deg_hbm,
              win_d, win_s, win_d2, win_s2, st_d0, st_s0, st_d1, st_s1,
              d0a, d0b, d0c, d0d, d1a, d1b, d1c, d1d,
              cnt_v, call_v, sent_buf_d, sent_buf_s,
              semda, semdb, semsa, semsb):
    c = lax.axis_index("c")
    s = lax.axis_index("s")
    lo0 = (c * NRC + s) * R
    lo1 = (c * NRC + 16 + s) * R
    zero_f = jnp.zeros((16,), jnp.float32)
    ones_f = jnp.ones((16,), jnp.float32)
    lane = lax.iota(jnp.int32, 16)

    deg0s = (d0a, d0b, d0c, d0d)
    deg1s = (d1a, d1b, d1c, d1d)

    @pl.loop(0, R + 16, step=16)
    def _(i):
        for k in range(4):
            deg0s[k][pl.ds(i, 16)] = zero_f
            deg1s[k][pl.ds(i, 16)] = zero_f

    def _hist_win(buf):
        def vb4(j, carry):
            for k in range(4):
                v = buf[pl.ds(pl.multiple_of(j * 64, 16) + k * 16, 16)]
                v0 = v - lo0
                v1 = v - lo1
                m0 = (v0 >= 0) & (v0 < R)
                m1 = (v1 >= 0) & (v1 < R)
                plsc.addupdate_scatter(deg0s[k], [jnp.where(m0, v0, R + lane)], ones_f)
                plsc.addupdate_scatter(deg1s[k], [jnp.where(m1, v1, R + lane)], ones_f)
            return carry

        lax.fori_loop(0, WSC // 64, vb4, _i0)

    def _dwin(w):
        return dst_hbm.at[pl.ds(pl.multiple_of(w * WSC, 8), WSC)]

    def _swin(w):
        return src_hbm.at[pl.ds(pl.multiple_of(w * WSC, 8), WSC)]

    pltpu.async_copy(_dwin(0), win_d, semda)

    def pair1(i, _):
        w = i * 2
        pltpu.async_copy(_dwin(w + 1), win_d2, semdb)
        pltpu.make_async_copy(_dwin(w), win_d, semda).wait()
        _hist_win(win_d)

        @pl.when(w + 2 < NWIN)
        def _():
            pltpu.async_copy(_dwin(w + 2), win_d, semda)

        pltpu.make_async_copy(_dwin(w + 1), win_d2, semdb).wait()
        _hist_win(win_d2)
        return _i0

    lax.fori_loop(0, NWIN // 2, pair1, _i0)
    acc0 = jnp.zeros((16,), jnp.float32)
    acc1 = jnp.zeros((16,), jnp.float32)
    for i in range(R // 16):
        m0v = (d0a[pl.ds(i * 16, 16)] + d0b[pl.ds(i * 16, 16)]
               + d0c[pl.ds(i * 16, 16)] + d0d[pl.ds(i * 16, 16)])
        m1v = (d1a[pl.ds(i * 16, 16)] + d1b[pl.ds(i * 16, 16)]
               + d1c[pl.ds(i * 16, 16)] + d1d[pl.ds(i * 16, 16)])
        d0a[pl.ds(i * 16, 16)] = m0v
        d1a[pl.ds(i * 16, 16)] = m1v
        acc0 = acc0 + m0v
        acc1 = acc1 + m1v
    c0 = jnp.sum(acc0).astype(jnp.int32)
    c1 = jnp.sum(acc1).astype(jnp.int32)
    pc0 = _ceil512(c0)
    pc1 = _ceil512(c1)

    cnt_v[pl.ds(0, 16)] = jnp.full((16,), pc0, jnp.int32)
    pltpu.sync_copy(cnt_v, counts_hbm.at[pl.ds(pl.multiple_of((c * NRC + s) * 16, 16), 16)])
    cnt_v[pl.ds(0, 16)] = jnp.full((16,), pc1, jnp.int32)
    pltpu.sync_copy(cnt_v, counts_hbm.at[pl.ds(pl.multiple_of((c * NRC + s + 16) * 16, 16), 16)])
    plsc.subcore_barrier()

    pltpu.sync_copy(counts_hbm.at[pl.ds(pl.multiple_of(c * NRC * 16, 16), NRC * 16)], call_v)
    base0 = _i0
    base1 = _i0
    for i in range(NRC):
        pci = call_v[pl.ds(i * 16, 16)][0]
        base0 = base0 + jnp.where(i < s, pci, 0)
        base1 = base1 + jnp.where(i < s + 16, pci, 0)
    base0 = pl.multiple_of(base0, 512)
    base1 = pl.multiple_of(base1, 512)

    def _place_win(bufd, bufs_, carry):
        def vb4(j, fc):
            f0, f1, d0, d1 = fc
            for k in range(4):
                off = pl.multiple_of(j * 64, 16) + k * 16
                v = bufd[pl.ds(off, 16)]
                u = bufs_[pl.ds(off, 16)]
                v0 = v - lo0
                v1 = v - lo1
                m0 = (v0 >= 0) & (v0 < R)
                m1 = (v1 >= 0) & (v1 < R)
                plsc.store_compressed(st_d0.at[pl.ds(f0, 16)], v0, mask=m0)
                plsc.store_compressed(st_s0.at[pl.ds(f0, 16)], u, mask=m0)
                plsc.store_compressed(st_d1.at[pl.ds(f1, 16)], v1, mask=m1)
                plsc.store_compressed(st_s1.at[pl.ds(f1, 16)], u, mask=m1)
                f0 = f0 + plsc.all_reduce_population_count(m0)[0]
                f1 = f1 + plsc.all_reduce_population_count(m1)[0]

            def flush0(args):
                f, d = args
                pltpu.sync_copy(st_d0.at[pl.ds(0, SB)],
                                ebd_hbm.at[pl.ds(pl.multiple_of(c * EPC + base0 + d, 64), SB)])
                pltpu.sync_copy(st_s0.at[pl.ds(0, SB)],
                                ebs_hbm.at[pl.ds(pl.multiple_of(c * EPC + base0 + d, 64), SB)])
                for t in range(8):
                    st_d0[pl.ds(t * 16, 16)] = st_d0[pl.ds(SB + t * 16, 16)]
                    st_s0[pl.ds(t * 16, 16)] = st_s0[pl.ds(SB + t * 16, 16)]
                return f - SB, d + SB

            def flush1(args):
                f, d = args
                pltpu.sync_copy(st_d1.at[pl.ds(0, SB)],
                                ebd_hbm.at[pl.ds(pl.multiple_of(c * EPC + base1 + d, 64), SB)])
                pltpu.sync_copy(st_s1.at[pl.ds(0, SB)],
                                ebs_hbm.at[pl.ds(pl.multiple_of(c * EPC + base1 + d, 64), SB)])
                for t in range(8):
                    st_d1[pl.ds(t * 16, 16)] = st_d1[pl.ds(SB + t * 16, 16)]
                    st_s1[pl.ds(t * 16, 16)] = st_s1[pl.ds(SB + t * 16, 16)]
                return f - SB, d + SB

            f0, d0 = lax.cond(f0 >= SB, flush0, lambda a_: a_, (f0, d0))
            f1, d1 = lax.cond(f1 >= SB, flush1, lambda a_: a_, (f1, d1))
            return f0, f1, d0, d1

        return lax.fori_loop(0, WSC // 64, vb4, carry)

    pltpu.async_copy(_dwin(0), win_d, semda)
    pltpu.async_copy(_swin(0), win_s, semsa)

    def pair2(i, fc):
        w = i * 2
        pltpu.async_copy(_dwin(w + 1), win_d2, semdb)
        pltpu.async_copy(_swin(w + 1), win_s2, semsb)
        pltpu.make_async_copy(_dwin(w), win_d, semda).wait()
        pltpu.make_async_copy(_swin(w), win_s, semsa).wait()
        fc = _place_win(win_d, win_s, fc)

        @pl.when(w + 2 < NWIN)
        def _():
            pltpu.async_copy(_dwin(w + 2), win_d, semda)
            pltpu.async_copy(_swin(w + 2), win_s, semsa)

        pltpu.make_async_copy(_dwin(w + 1), win_d2, semdb).wait()
        pltpu.make_async_copy(_swin(w + 1), win_s2, semsb).wait()
        fc = _place_win(win_d2, win_s2, fc)
        return fc

    f0, f1, d0, d1 = lax.fori_loop(0, NWIN // 2, pair2, (_i0, _i0, _i0, _i0))

    sent_d = jnp.zeros((16,), jnp.int32)
    sent_s = jnp.full((16,), N, jnp.int32)
    all_m = lane < 16
    for k in range(4):
        plsc.store_compressed(st_d0.at[pl.ds(f0 + k * 16, 16)], sent_d, mask=all_m)
        plsc.store_compressed(st_s0.at[pl.ds(f0 + k * 16, 16)], sent_s, mask=all_m)
        plsc.store_compressed(st_d1.at[pl.ds(f1 + k * 16, 16)], sent_d, mask=all_m)
        plsc.store_compressed(st_s1.at[pl.ds(f1 + k * 16, 16)], sent_s, mask=all_m)

    def fin0(k, _):
        pltpu.sync_copy(st_d0.at[pl.ds(k * 64, 64)],
                        ebd_hbm.at[pl.ds(pl.multiple_of(c * EPC + base0 + d0 + k * 64, 64), 64)])
        pltpu.sync_copy(st_s0.at[pl.ds(k * 64, 64)],
                        ebs_hbm.at[pl.ds(pl.multiple_of(c * EPC + base0 + d0 + k * 64, 64), 64)])
        return _i0

    def fin1(k, _):
        pltpu.sync_copy(st_d1.at[pl.ds(k * 64, 64)],
                        ebd_hbm.at[pl.ds(pl.multiple_of(c * EPC + base1 + d1 + k * 64, 64), 64)])
        pltpu.sync_copy(st_s1.at[pl.ds(k * 64, 64)],
                        ebs_hbm.at[pl.ds(pl.multiple_of(c * EPC + base1 + d1 + k * 64, 64), 64)])
        return _i0

    lax.fori_loop(0, (f0 + 63) // 64, fin0, _i0)
    lax.fori_loop(0, (f1 + 63) // 64, fin1, _i0)

    for k in range(4):
        sent_buf_d[pl.ds(k * 16, 16)] = sent_d
        sent_buf_s[pl.ds(k * 16, 16)] = jnp.full((16,), N + k * 16, jnp.int32) + lane

    cov0 = d0 + ((f0 + 63) // 64) * 64
    cov1 = d1 + ((f1 + 63) // 64) * 64

    def pad0(k, _):
        pltpu.sync_copy(sent_buf_d,
                        ebd_hbm.at[pl.ds(pl.multiple_of(c * EPC + base0 + cov0 + k * 64, 64), 64)])
        pltpu.sync_copy(sent_buf_s,
                        ebs_hbm.at[pl.ds(pl.multiple_of(c * EPC + base0 + cov0 + k * 64, 64), 64)])
        return _i0

    def pad1(k, _):
        pltpu.sync_copy(sent_buf_d,
                        ebd_hbm.at[pl.ds(pl.multiple_of(c * EPC + base1 + cov1 + k * 64, 64), 64)])
        pltpu.sync_copy(sent_buf_s,
                        ebs_hbm.at[pl.ds(pl.multiple_of(c * EPC + base1 + cov1 + k * 64, 64), 64)])
        return _i0

    lax.fori_loop(0, (pc0 - cov0) // 64, pad0, _i0)
    lax.fori_loop(0, (pc1 - cov1) // 64, pad1, _i0)

    pltpu.sync_copy(d0a.at[pl.ds(0, R)], deg_hbm.at[pl.ds(pl.multiple_of(lo0, 16), R)])
    pltpu.sync_copy(d1a.at[pl.ds(0, R)], deg_hbm.at[pl.ds(pl.multiple_of(lo1, 16), R)])


def _k12(dst, src):
    return pl.kernel(
        _k12_body,
        out_type=(
            jax.ShapeDtypeStruct((NC * NRC * 16,), jnp.int32),
            jax.ShapeDtypeStruct((NC * EPC,), jnp.int32),
            jax.ShapeDtypeStruct((NC * EPC,), jnp.int32),
            jax.ShapeDtypeStruct((NPAD,), jnp.float32),
        ),
        mesh=_mesh,
        scratch_types=[
            pltpu.VMEM((WSC,), jnp.int32),
            pltpu.VMEM((WSC,), jnp.int32),
            pltpu.VMEM((WSC,), jnp.int32),
            pltpu.VMEM((WSC,), jnp.int32),
            pltpu.VMEM((STG,), jnp.int32),
            pltpu.VMEM((STG,), jnp.int32),
            pltpu.VMEM((STG,), jnp.int32),
            pltpu.VMEM((STG,), jnp.int32),
            pltpu.VMEM((R + 16,), jnp.float32),
            pltpu.VMEM((R + 16,), jnp.float32),
            pltpu.VMEM((R + 16,), jnp.float32),
            pltpu.VMEM((R + 16,), jnp.float32),
            pltpu.VMEM((R + 16,), jnp.float32),
            pltpu.VMEM((R + 16,), jnp.float32),
            pltpu.VMEM((R + 16,), jnp.float32),
            pltpu.VMEM((R + 16,), jnp.float32),
            pltpu.VMEM((16,), jnp.int32),
            pltpu.VMEM((NRC * 16,), jnp.int32),
            pltpu.VMEM((64,), jnp.int32),
            pltpu.VMEM((64,), jnp.int32),
            pltpu.SemaphoreType.DMA,
            pltpu.SemaphoreType.DMA,
            pltpu.SemaphoreType.DMA,
            pltpu.SemaphoreType.DMA,
        ],
        compiler_params=_cp,
    )(dst, src)


def _k3_body(hp_hbm, counts_hbm, ebd_hbm, ebs_hbm, s_hbm,
             call_v, acc_v, sidx, didx, rows0, rows1, sem0, sem1):
    c = lax.axis_index("c")
    s = lax.axis_index("s")
    zero_f = jnp.zeros((16,), jnp.float32)

    pltpu.sync_copy(counts_hbm.at[pl.ds(pl.multiple_of(c * NRC * 16, 16), NRC * 16)], call_v)

    base0 = _i0
    base1 = _i0
    for i in range(NRC):
        pci = call_v[pl.ds(i * 16, 16)][0]
        base0 = base0 + jnp.where(i < s, pci, 0)
        base1 = base1 + jnp.where(i < s + 16, pci, 0)
    base0 = pl.multiple_of(base0, 512)
    base1 = pl.multiple_of(base1, 512)
    pc0 = call_v[pl.ds(pl.multiple_of(s * 16, 16), 16)][0]
    pc1 = call_v[pl.ds(pl.multiple_of((s + 16) * 16, 16), 16)][0]

    bufs = (rows0, rows1)
    sems = (sem0, sem1)

    for p in range(2):
        base = base0 if p == 0 else base1
        pc = pc0 if p == 0 else pc1
        rng = c * NRC + p * 16 + s

        @pl.loop(0, R)
        def _(i):
            for f in range(8):
                acc_v[i, pl.ds(f * 16, 16)] = zero_f

        def chunk(ch, _):
            coff = pl.multiple_of(c * EPC + base + ch * 512, 512)
            pltpu.sync_copy(ebs_hbm.at[pl.ds(coff, 512)], sidx)
            pltpu.sync_copy(ebd_hbm.at[pl.ds(coff, 512)], didx)
            h = [None] * 8
            h[0] = pltpu.async_copy(hp_hbm.at[sidx.at[pl.ds(0, 64)]], rows0, sem0)
            for w in range(8):
                if w < 7:
                    h[w + 1] = pltpu.async_copy(
                        hp_hbm.at[sidx.at[pl.ds((w + 1) * 64, 64)]],
                        bufs[(w + 1) % 2], sems[(w + 1) % 2])
                h[w].wait()
                rows = bufs[w % 2]

                @pl.loop(0, 64, step=16)
                def _(j, w=w, rows=rows):
                    dv = didx[pl.ds(pl.multiple_of(w * 64 + j, 16), 16)]
                    for l in range(16):
                        d = dv[l]
                        for f in range(8):
                            plsc.addupdate(
                                acc_v.at[d, pl.ds(f * 16, 16)],
                                rows[j + l, pl.ds(f * 16, 16)],
                            )

            return _i0

        lax.fori_loop(0, pc // 512, chunk, _i0)
        pltpu.sync_copy(acc_v, s_hbm.at[pl.ds(pl.multiple_of(rng * R, 16), R), :])


def _k3(hp, counts, ebd, ebs):
    return pl.kernel(
        _k3_body,
        out_type=jax.ShapeDtypeStruct((NPAD, D), jnp.float32),
        mesh=_mesh,
        scratch_types=[
            pltpu.VMEM((NRC * 16,), jnp.int32),
            pltpu.VMEM((R, D), jnp.float32),
            pltpu.VMEM((512,), jnp.int32),
            pltpu.VMEM((512,), jnp.int32),
            pltpu.VMEM((WG, D), jnp.float32),
            pltpu.VMEM((WG, D), jnp.float32),
            pltpu.SemaphoreType.DMA,
            pltpu.SemaphoreType.DMA,
        ],
    )(hp, counts, ebd, ebs)


_BM = 784
_GRID = NPAD // _BM


def _mm1_k(z_ref, w_ref, o_ref):
    o_ref[...] = lax.dot_general(
        z_ref[...], w_ref[...], (((1,), (0,)), ((), ())),
        precision=lax.Precision.HIGHEST, preferred_element_type=jnp.float32)


def _mm1(zp, W1):
    return pl.pallas_call(
        _mm1_k,
        out_shape=jax.ShapeDtypeStruct((NPAD, D), jnp.float32),
        grid=(_GRID,),
        in_specs=[
            pl.BlockSpec((_BM, 64), lambda i: (i, 0)),
            pl.BlockSpec((64, D), lambda i: (0, 0)),
        ],
        out_specs=pl.BlockSpec((_BM, D), lambda i: (i, 0)),
    )(zp, W1)


def _scale_k(deg_ref, h_ref, o_ref):
    dinv = lax.rsqrt(deg_ref[...] + 1.0)
    o_ref[...] = dinv * h_ref[...]


def _scale(deg2, h):
    return pl.pallas_call(
        _scale_k,
        out_shape=jax.ShapeDtypeStruct((NPAD, D), jnp.float32),
        grid=(_GRID,),
        in_specs=[
            pl.BlockSpec((_BM, 1), lambda i: (i, 0)),
            pl.BlockSpec((_BM, D), lambda i: (i, 0)),
        ],
        out_specs=pl.BlockSpec((_BM, D), lambda i: (i, 0)),
    )(deg2, h)


def _c1_k(deg_ref, s_ref, h1_ref, w2_ref, b1_ref, h2_ref, hp2_ref):
    i = pl.program_id(0)
    dinv = lax.rsqrt(deg_ref[...] + 1.0)
    t = dinv * s_ref[...] + dinv * dinv * h1_ref[...] + b1_ref[...]
    h = jnp.maximum(t, 0.0)
    h2 = lax.dot_general(
        h, w2_ref[...], (((1,), (0,)), ((), ())),
        precision=lax.Precision.HIGHEST, preferred_element_type=jnp.float32)
    h2_ref[...] = h2
    rid = i * _BM + lax.broadcasted_iota(jnp.int32, (_BM, 1), 0)
    hp2_ref[...] = jnp.where(rid < N, dinv * h2, 0.0)


def _c1(deg2, s1, h1, W2, b1):
    return pl.pallas_call(
        _c1_k,
        out_shape=(
            jax.ShapeDtypeStruct((NPAD, D), jnp.float32),
            jax.ShapeDtypeStruct((NPAD, D), jnp.float32),
        ),
        grid=(_GRID,),
        in_specs=[
            pl.BlockSpec((_BM, 1), lambda i: (i, 0)),
            pl.BlockSpec((_BM, D), lambda i: (i, 0)),
            pl.BlockSpec((_BM, D), lambda i: (i, 0)),
            pl.BlockSpec((D, D), lambda i: (0, 0)),
            pl.BlockSpec((1, D), lambda i: (0, 0)),
        ],
        out_specs=(
            pl.BlockSpec((_BM, D), lambda i: (i, 0)),
            pl.BlockSpec((_BM, D), lambda i: (i, 0)),
        ),
    )(deg2, s1, h1, W2, b1)


_BO = 2000


def _c2_k(deg_ref, s_ref, h2_ref, b2_ref, o_ref):
    dinv = lax.rsqrt(deg_ref[...] + 1.0)
    o_ref[...] = dinv * s_ref[...] + dinv * dinv * h2_ref[...] + b2_ref[...]


def _c2(deg2, s2, h2, b2):
    return pl.pallas_call(
        _c2_k,
        out_shape=jax.ShapeDtypeStruct((N, D), jnp.float32),
        grid=(N // _BO,),
        in_specs=[
            pl.BlockSpec((_BO, 1), lambda i: (i, 0)),
            pl.BlockSpec((_BO, D), lambda i: (i, 0)),
            pl.BlockSpec((_BO, D), lambda i: (i, 0)),
            pl.BlockSpec((1, D), lambda i: (0, 0)),
        ],
        out_specs=pl.BlockSpec((_BO, D), lambda i: (i, 0)),
    )(deg2, s2, h2, b2)


def kernel(z, edge_index, W1, b1, W2, b2):
    src = edge_index[0]
    dst = edge_index[1]
    zp = jnp.pad(z, ((0, NPAD - N), (0, 0)))

    counts, ebd, ebs, deg = _k12(dst, src)
    deg2 = deg[:, None]

    h1 = _mm1(zp, W1)
    hp1 = _scale(deg2, h1)
    s1 = _k3(hp1, counts, ebd, ebs)
    h2, hp2 = _c1(deg2, s1, h1, W2, b1[None, :])
    s2 = _k3(hp2, counts, ebd, ebs)
    return _c2(deg2, s2, h2, b2[None, :])
